# Initial kernel scaffold; baseline (speedup 1.0000x reference)
import jax
import jax.numpy as jnp
from jax.experimental import pallas as pl


def kernel(x, W1, g1, b1, W2, g2, b2, W3, g3, b3, W4, g4, b4, Wm1, gm1, bm1, Wm2, gm2, bm2, Wfc, bfc):
    raise NotImplementedError("write your pallas kernel here")


# speedup vs baseline: 11.2771x; 11.2771x over previous
import functools
import math

import jax
import jax.numpy as jnp
from jax import lax
from jax.experimental import pallas as pl
from jax.experimental.pallas import tpu as pltpu
from jax.experimental.pallas import tpu_sc as plsc

K = 20
EPS = 1e-5
N = 1024
NB = 8
SCALE = float(1.0 / math.sqrt(1.0 + EPS))
NWORKERS = 32
CH = 8


def _edge_core(xt, wn_ref, wq_ref, b_ref, idx_ref, p_ref, q_ref, d_ref):
    boff = pl.program_id(0) * N
    s = lax.dot_general(xt, xt, (((1,), (1,)), ((), ())),
                        preferred_element_type=jnp.float32)
    sq = jnp.sum(xt * xt, axis=1)
    d_ref[...] = 2.0 * s - sq[:, None] - sq[None, :]
    iota = lax.broadcasted_iota(jnp.int32, (N, N), 1)

    def body(j, m):
        d = d_ref[...]
        cand = jnp.where(d >= m, iota, N)
        amin = jnp.min(cand, axis=1)
        idx_ref[0, pl.ds(j, 1), :] = (amin + boff)[None, :]
        d = jnp.where(iota == amin[:, None], -jnp.inf, d)
        d_ref[...] = d
        return jnp.max(d, axis=1, keepdims=True)

    m0 = jnp.max(d_ref[...], axis=1, keepdims=True)
    lax.fori_loop(0, K, body, m0)

    p_ref[0] = lax.dot_general(xt, wn_ref[...], (((1,), (1,)), ((), ())),
                               preferred_element_type=jnp.float32)
    q_ref[0] = lax.dot_general(xt, wq_ref[...], (((1,), (1,)), ((), ())),
                               preferred_element_type=jnp.float32) + b_ref[...]


def _edge_pre_body(xt_ref, wn_ref, wq_ref, b_ref, idx_ref, p_ref, q_ref, d_ref):
    _edge_core(xt_ref[0], wn_ref, wq_ref, b_ref, idx_ref, p_ref, q_ref, d_ref)


def _edge_mid_body(g_ref, qp_ref, wn_ref, wq_ref, b_ref,
                   idx_ref, p_ref, q_ref, avg_ref, mx_ref, d_ref):
    z = g_ref[0] + qp_ref[0]
    xt = jnp.where(z > 0, z, 0.2 * z)
    avg_ref[0, 0, :] = jnp.mean(xt, axis=0)
    mx_ref[0, 0, :] = jnp.max(xt, axis=0)
    _edge_core(xt, wn_ref, wq_ref, b_ref, idx_ref, p_ref, q_ref, d_ref)


def _tail_body(g_ref, qp_ref, avg_ref, mx_ref):
    z = g_ref[0] + qp_ref[0]
    xt = jnp.where(z > 0, z, 0.2 * z)
    avg_ref[0, 0, :] = jnp.mean(xt, axis=0)
    mx_ref[0, 0, :] = jnp.max(xt, axis=0)


def _edge_pre(xt, wn, wq, beta):
    pout = wn.shape[0]
    out = wq.shape[0]
    c = xt.shape[2]
    return pl.pallas_call(
        _edge_pre_body,
        grid=(NB,),
        in_specs=[pl.BlockSpec((1, N, c), lambda b: (b, 0, 0)),
                  pl.BlockSpec((pout, c), lambda b: (0, 0)),
                  pl.BlockSpec((out, c), lambda b: (0, 0)),
                  pl.BlockSpec((1, out), lambda b: (0, 0))],
        out_specs=[pl.BlockSpec((1, K, N), lambda b: (b, 0, 0)),
                   pl.BlockSpec((1, N, pout), lambda b: (b, 0, 0)),
                   pl.BlockSpec((1, N, out), lambda b: (b, 0, 0))],
        out_shape=[jax.ShapeDtypeStruct((NB, K, N), jnp.int32),
                   jax.ShapeDtypeStruct((NB, N, pout), jnp.float32),
                   jax.ShapeDtypeStruct((NB, N, out), jnp.float32)],
        scratch_shapes=[pltpu.VMEM((N, N), jnp.float32)],
    )(xt, wn, wq, beta.reshape(1, out))


def _edge_mid(g, qp, wn, wq, beta):
    pout = wn.shape[0]
    out = wq.shape[0]
    cin = qp.shape[2]
    return pl.pallas_call(
        _edge_mid_body,
        grid=(NB,),
        in_specs=[pl.BlockSpec((1, N, cin), lambda b: (b, 0, 0)),
                  pl.BlockSpec((1, N, cin), lambda b: (b, 0, 0)),
                  pl.BlockSpec((pout, cin), lambda b: (0, 0)),
                  pl.BlockSpec((out, cin), lambda b: (0, 0)),
                  pl.BlockSpec((1, out), lambda b: (0, 0))],
        out_specs=[pl.BlockSpec((1, K, N), lambda b: (b, 0, 0)),
                   pl.BlockSpec((1, N, pout), lambda b: (b, 0, 0)),
                   pl.BlockSpec((1, N, out), lambda b: (b, 0, 0)),
                   pl.BlockSpec((1, 1, cin), lambda b: (b, 0, 0)),
                   pl.BlockSpec((1, 1, cin), lambda b: (b, 0, 0))],
        out_shape=[jax.ShapeDtypeStruct((NB, K, N), jnp.int32),
                   jax.ShapeDtypeStruct((NB, N, pout), jnp.float32),
                   jax.ShapeDtypeStruct((NB, N, out), jnp.float32),
                   jax.ShapeDtypeStruct((NB, 1, cin), jnp.float32),
                   jax.ShapeDtypeStruct((NB, 1, cin), jnp.float32)],
        scratch_shapes=[pltpu.VMEM((N, N), jnp.float32)],
    )(g, qp, wn, wq, beta.reshape(1, out))


def _tail(g, qp):
    cin = qp.shape[2]
    return pl.pallas_call(
        _tail_body,
        grid=(NB,),
        in_specs=[pl.BlockSpec((1, N, cin), lambda b: (b, 0, 0)),
                  pl.BlockSpec((1, N, cin), lambda b: (b, 0, 0))],
        out_specs=[pl.BlockSpec((1, 1, cin), lambda b: (b, 0, 0)),
                   pl.BlockSpec((1, 1, cin), lambda b: (b, 0, 0))],
        out_shape=[jax.ShapeDtypeStruct((NB, 1, cin), jnp.float32),
                   jax.ShapeDtypeStruct((NB, 1, cin), jnp.float32)],
    )(g, qp)


def _sc_gather_max(idx, p, out):
    bn, pout = p.shape
    ppw = bn // NWORKERS
    mesh = plsc.VectorSubcoreMesh(core_axis_name="c", subcore_axis_name="s")

    grp = 128

    @functools.partial(
        pl.kernel,
        out_type=jax.ShapeDtypeStruct((bn, out), jnp.float32),
        mesh=mesh,
        scratch_types=[pltpu.VMEM((K, grp), jnp.int32),
                       pltpu.VMEM((K * CH, pout), jnp.float32),
                       pltpu.VMEM((CH, out), jnp.float32),
                       pltpu.SemaphoreType.DMA],
    )
    def sck(idx_hbm, p_hbm, g_hbm, idx_v, rows_v, acc_v, sem):
        wid = lax.axis_index("s") * 2 + lax.axis_index("c")
        base = wid * ppw

        @pl.loop(0, ppw // grp)
        def _group(gi):
            gbase = base + gi * grp
            b = gbase // N
            n0 = gbase - b * N
            pltpu.sync_copy(idx_hbm.at[b, :, pl.ds(n0, grp)], idx_v)

            @pl.loop(0, grp // CH)
            def _chunk(ci):
                w0 = ci * CH
                cps = [pltpu.async_copy(p_hbm.at[idx_v.at[j, pl.ds(w0, CH)]],
                                        rows_v.at[pl.ds(j * CH, CH)], sem)
                       for j in range(K)]
                for cp in cps:
                    cp.wait()

                @pl.loop(0, out // 16)
                def _cols(c):
                    co = pl.multiple_of(c * 16, 16)
                    for w in range(CH):
                        acc = rows_v[w, pl.ds(co, 16)]
                        for j in range(1, K):
                            acc = jnp.maximum(acc, rows_v[j * CH + w, pl.ds(co, 16)])
                        acc_v[w, pl.ds(co, 16)] = acc

                pltpu.sync_copy(acc_v, g_hbm.at[pl.ds(gbase + w0, CH)])

    return sck(idx, p)


def _head_body(f_ref, w1_ref, b1_ref, w2_ref, b2_ref, w3_ref, b3_ref, o_ref):
    dn = (((1,), (1,)), ((), ()))
    h = lax.dot_general(f_ref[...], w1_ref[...], dn,
                        preferred_element_type=jnp.float32) + b1_ref[...]
    h = jnp.maximum(h, 0.0)
    h = lax.dot_general(h, w2_ref[...], dn,
                        preferred_element_type=jnp.float32) + b2_ref[...]
    h = jnp.maximum(h, 0.0)
    o_ref[...] = lax.dot_general(h, w3_ref[...], dn,
                                 preferred_element_type=jnp.float32) + b3_ref[...]


def _head(f, w1, b1, w2, b2, w3, b3):
    return pl.pallas_call(
        _head_body,
        out_shape=jax.ShapeDtypeStruct((NB, w3.shape[0]), jnp.float32),
    )(f, w1, b1.reshape(1, -1), w2, b2.reshape(1, -1), w3, b3.reshape(1, -1))


def _prep(W, g, cin):
    wn, wc = W[:, :cin], W[:, cin:]
    s = (g * SCALE)[:, None]
    return wn * s, (wc - wn) * s


def kernel(x, W1, g1, b1, W2, g2, b2, W3, g3, b3, W4, g4, b4,
           Wm1, gm1, bm1, Wm2, gm2, bm2, Wfc, bfc):
    wn1, wq1 = _prep(W1, g1, 3)
    wn2, wq2 = _prep(W2, g2, 64)
    wn3, wq3 = _prep(W3, g3, 64)
    wn4, wq4 = _prep(W4, g4, 128)
    wn1 = jnp.pad(wn1, ((0, 64), (0, 0)))
    wn2 = jnp.pad(wn2, ((0, 64), (0, 0)))
    xt0 = jnp.transpose(x, (0, 2, 1))
    xt0 = jnp.pad(xt0, ((0, 0), (0, 0), (0, 5)))
    wn1 = jnp.pad(wn1, ((0, 0), (0, 5)))
    wq1 = jnp.pad(wq1, ((0, 0), (0, 5)))

    idx1, p1, q1 = _edge_pre(xt0, wn1, wq1, b1)
    gm1_ = _sc_gather_max(idx1, p1.reshape(NB * N, -1), 64)

    idx2, p2, q2, avg1, mx1 = _edge_mid(gm1_.reshape(NB, N, -1), q1, wn2, wq2, b2)
    gm2_ = _sc_gather_max(idx2, p2.reshape(NB * N, -1), 64)

    idx3, p3, q3, avg2, mx2 = _edge_mid(gm2_.reshape(NB, N, -1), q2, wn3, wq3, b3)
    gm3_ = _sc_gather_max(idx3, p3.reshape(NB * N, -1), 128)

    idx4, p4, q4, avg3, mx3 = _edge_mid(gm3_.reshape(NB, N, -1), q3, wn4, wq4, b4)
    gm4_ = _sc_gather_max(idx4, p4.reshape(NB * N, -1), 256)

    avg4, mx4 = _tail(gm4_.reshape(NB, N, -1), q4)

    f = jnp.concatenate(
        [avg1[:, 0], avg2[:, 0], avg3[:, 0], avg4[:, 0],
         mx1[:, 0], mx2[:, 0], mx3[:, 0], mx4[:, 0]], axis=1)

    w1e = Wm1 * (gm1 * SCALE)[:, None]
    w2e = Wm2 * (gm2 * SCALE)[:, None]
    return _head(f, w1e, bm1, w2e, bm2, Wfc, bfc)


# speedup vs baseline: 16.4437x; 1.4581x over previous
import functools
import math

import jax
import jax.numpy as jnp
from jax import lax
from jax.experimental import pallas as pl
from jax.experimental.pallas import tpu as pltpu
from jax.experimental.pallas import tpu_sc as plsc

K = 20
EPS = 1e-5
N = 1024
NB = 8
SCALE = float(1.0 / math.sqrt(1.0 + EPS))
NWORKERS = 32
CH = 8


def _edge_core(xt, wn_ref, wq_ref, b_ref, idx_ref, p_ref, q_ref, d_ref):
    boff = pl.program_id(0) * N
    s = lax.dot_general(xt, xt, (((1,), (1,)), ((), ())),
                        preferred_element_type=jnp.float32)
    sq = jnp.sum(xt * xt, axis=1)
    d_ref[...] = 2.0 * s - (sq[:, None] + sq[None, :])
    iota0 = lax.broadcasted_iota(jnp.int32, (N, N), 0)

    def body(j, m):
        d = d_ref[...]
        cand = jnp.where(d >= m, iota0, N)
        amin = jnp.min(cand, axis=0, keepdims=True)
        idx_ref[0, pl.ds(j, 1), :] = amin + boff
        d = jnp.where(iota0 == amin, -jnp.inf, d)
        d_ref[...] = d
        return jnp.max(d, axis=0, keepdims=True)

    m0 = jnp.max(d_ref[...], axis=0, keepdims=True)
    lax.fori_loop(0, K, body, m0)

    p_ref[0] = lax.dot_general(xt, wn_ref[...], (((1,), (1,)), ((), ())),
                               preferred_element_type=jnp.float32)
    q_ref[0] = lax.dot_general(xt, wq_ref[...], (((1,), (1,)), ((), ())),
                               preferred_element_type=jnp.float32) + b_ref[...]


def _edge_pre_body(xt_ref, wn_ref, wq_ref, b_ref, idx_ref, p_ref, q_ref, d_ref):
    _edge_core(xt_ref[0], wn_ref, wq_ref, b_ref, idx_ref, p_ref, q_ref, d_ref)


def _edge_mid_body(g_ref, qp_ref, wn_ref, wq_ref, b_ref,
                   idx_ref, p_ref, q_ref, avg_ref, mx_ref, d_ref):
    z = g_ref[0] + qp_ref[0]
    xt = jnp.where(z > 0, z, 0.2 * z)
    avg_ref[0, 0, :] = jnp.mean(xt, axis=0)
    mx_ref[0, 0, :] = jnp.max(xt, axis=0)
    _edge_core(xt, wn_ref, wq_ref, b_ref, idx_ref, p_ref, q_ref, d_ref)


def _tail_body(g_ref, qp_ref, avg_ref, mx_ref):
    z = g_ref[0] + qp_ref[0]
    xt = jnp.where(z > 0, z, 0.2 * z)
    avg_ref[0, 0, :] = jnp.mean(xt, axis=0)
    mx_ref[0, 0, :] = jnp.max(xt, axis=0)


def _edge_pre(xt, wn, wq, beta):
    pout = wn.shape[0]
    out = wq.shape[0]
    c = xt.shape[2]
    return pl.pallas_call(
        _edge_pre_body,
        grid=(NB,),
        in_specs=[pl.BlockSpec((1, N, c), lambda b: (b, 0, 0)),
                  pl.BlockSpec((pout, c), lambda b: (0, 0)),
                  pl.BlockSpec((out, c), lambda b: (0, 0)),
                  pl.BlockSpec((1, out), lambda b: (0, 0))],
        out_specs=[pl.BlockSpec((1, K, N), lambda b: (b, 0, 0)),
                   pl.BlockSpec((1, N, pout), lambda b: (b, 0, 0)),
                   pl.BlockSpec((1, N, out), lambda b: (b, 0, 0))],
        out_shape=[jax.ShapeDtypeStruct((NB, K, N), jnp.int32),
                   jax.ShapeDtypeStruct((NB, N, pout), jnp.float32),
                   jax.ShapeDtypeStruct((NB, N, out), jnp.float32)],
        scratch_shapes=[pltpu.VMEM((N, N), jnp.float32)],
    )(xt, wn, wq, beta.reshape(1, out))


def _edge_mid(g, qp, wn, wq, beta):
    pout = wn.shape[0]
    out = wq.shape[0]
    cin = qp.shape[2]
    return pl.pallas_call(
        _edge_mid_body,
        grid=(NB,),
        in_specs=[pl.BlockSpec((1, N, cin), lambda b: (b, 0, 0)),
                  pl.BlockSpec((1, N, cin), lambda b: (b, 0, 0)),
                  pl.BlockSpec((pout, cin), lambda b: (0, 0)),
                  pl.BlockSpec((out, cin), lambda b: (0, 0)),
                  pl.BlockSpec((1, out), lambda b: (0, 0))],
        out_specs=[pl.BlockSpec((1, K, N), lambda b: (b, 0, 0)),
                   pl.BlockSpec((1, N, pout), lambda b: (b, 0, 0)),
                   pl.BlockSpec((1, N, out), lambda b: (b, 0, 0)),
                   pl.BlockSpec((1, 1, cin), lambda b: (b, 0, 0)),
                   pl.BlockSpec((1, 1, cin), lambda b: (b, 0, 0))],
        out_shape=[jax.ShapeDtypeStruct((NB, K, N), jnp.int32),
                   jax.ShapeDtypeStruct((NB, N, pout), jnp.float32),
                   jax.ShapeDtypeStruct((NB, N, out), jnp.float32),
                   jax.ShapeDtypeStruct((NB, 1, cin), jnp.float32),
                   jax.ShapeDtypeStruct((NB, 1, cin), jnp.float32)],
        scratch_shapes=[pltpu.VMEM((N, N), jnp.float32)],
    )(g, qp, wn, wq, beta.reshape(1, out))


def _tail(g, qp):
    cin = qp.shape[2]
    return pl.pallas_call(
        _tail_body,
        grid=(NB,),
        in_specs=[pl.BlockSpec((1, N, cin), lambda b: (b, 0, 0)),
                  pl.BlockSpec((1, N, cin), lambda b: (b, 0, 0))],
        out_specs=[pl.BlockSpec((1, 1, cin), lambda b: (b, 0, 0)),
                   pl.BlockSpec((1, 1, cin), lambda b: (b, 0, 0))],
        out_shape=[jax.ShapeDtypeStruct((NB, 1, cin), jnp.float32),
                   jax.ShapeDtypeStruct((NB, 1, cin), jnp.float32)],
    )(g, qp)


def _sc_gather_max(idx, p, out):
    bn, pout = p.shape
    ppw = bn // NWORKERS
    mesh = plsc.VectorSubcoreMesh(core_axis_name="c", subcore_axis_name="s")

    nchunks = ppw // CH

    @functools.partial(
        pl.kernel,
        out_type=jax.ShapeDtypeStruct((bn, out), jnp.float32),
        mesh=mesh,
        scratch_types=[pltpu.VMEM((K, ppw), jnp.int32),
                       pltpu.VMEM((K * CH, pout), jnp.float32),
                       pltpu.VMEM((K * CH, pout), jnp.float32),
                       pltpu.VMEM((CH, out), jnp.float32),
                       pltpu.VMEM((CH, out), jnp.float32),
                       pltpu.SemaphoreType.DMA,
                       pltpu.SemaphoreType.DMA,
                       pltpu.SemaphoreType.DMA,
                       pltpu.SemaphoreType.DMA],
    )
    def sck(idx_hbm, p_hbm, g_hbm, idx_v, rows0, rows1, acc0, acc1,
            gsem0, gsem1, ssem0, ssem1):
        wid = lax.axis_index("s") * 2 + lax.axis_index("c")
        base = wid * ppw
        b = base // N
        n0 = base - b * N
        pltpu.sync_copy(idx_hbm.at[b, :, pl.ds(n0, ppw)], idx_v)

        def fire(ci, rows, sem):
            w0 = ci * CH
            for j in range(K):
                pltpu.async_copy(p_hbm.at[idx_v.at[j, pl.ds(w0, CH)]],
                                 rows.at[pl.ds(j * CH, CH)], sem)

        def drain_gathers(rows, sem):
            for j in range(K):
                pltpu.make_async_copy(p_hbm.at[pl.ds(0, CH)],
                                      rows.at[pl.ds(j * CH, CH)], sem).wait()

        def compute_store(hi, ci, rows, acc, ssem):
            @pl.when(hi > 0)
            def _():
                pltpu.make_async_copy(acc, g_hbm.at[pl.ds(base, CH)], ssem).wait()

            @pl.loop(0, out // 16)
            def _cols(c):
                co = pl.multiple_of(c * 16, 16)
                for w in range(CH):
                    v = rows[w, pl.ds(co, 16)]
                    for j in range(1, K):
                        v = jnp.maximum(v, rows[j * CH + w, pl.ds(co, 16)])
                    acc[w, pl.ds(co, 16)] = v

            pltpu.async_copy(acc, g_hbm.at[pl.ds(base + ci * CH, CH)], ssem)

        fire(0, rows0, gsem0)

        @pl.loop(0, nchunks // 2)
        def _pair(hi):
            ci0 = 2 * hi
            fire(ci0 + 1, rows1, gsem1)
            drain_gathers(rows0, gsem0)
            compute_store(hi, ci0, rows0, acc0, ssem0)

            @pl.when(ci0 + 2 < nchunks)
            def _():
                fire(ci0 + 2, rows0, gsem0)

            drain_gathers(rows1, gsem1)
            compute_store(hi, ci0 + 1, rows1, acc1, ssem1)

        pltpu.make_async_copy(acc0, g_hbm.at[pl.ds(base, CH)], ssem0).wait()
        pltpu.make_async_copy(acc1, g_hbm.at[pl.ds(base, CH)], ssem1).wait()

    return sck(idx, p)


def _head_body(f_ref, w1_ref, b1_ref, w2_ref, b2_ref, w3_ref, b3_ref, o_ref):
    dn = (((1,), (1,)), ((), ()))
    h = lax.dot_general(f_ref[...], w1_ref[...], dn,
                        preferred_element_type=jnp.float32) + b1_ref[...]
    h = jnp.maximum(h, 0.0)
    h = lax.dot_general(h, w2_ref[...], dn,
                        preferred_element_type=jnp.float32) + b2_ref[...]
    h = jnp.maximum(h, 0.0)
    o_ref[...] = lax.dot_general(h, w3_ref[...], dn,
                                 preferred_element_type=jnp.float32) + b3_ref[...]


def _head(f, w1, b1, w2, b2, w3, b3):
    return pl.pallas_call(
        _head_body,
        out_shape=jax.ShapeDtypeStruct((NB, w3.shape[0]), jnp.float32),
    )(f, w1, b1.reshape(1, -1), w2, b2.reshape(1, -1), w3, b3.reshape(1, -1))


def _prep(W, g, cin):
    wn, wc = W[:, :cin], W[:, cin:]
    s = (g * SCALE)[:, None]
    return wn * s, (wc - wn) * s


def kernel(x, W1, g1, b1, W2, g2, b2, W3, g3, b3, W4, g4, b4,
           Wm1, gm1, bm1, Wm2, gm2, bm2, Wfc, bfc):
    wn1, wq1 = _prep(W1, g1, 3)
    wn2, wq2 = _prep(W2, g2, 64)
    wn3, wq3 = _prep(W3, g3, 64)
    wn4, wq4 = _prep(W4, g4, 128)
    wn1 = jnp.pad(wn1, ((0, 64), (0, 0)))
    wn2 = jnp.pad(wn2, ((0, 64), (0, 0)))
    xt0 = jnp.transpose(x, (0, 2, 1))
    xt0 = jnp.pad(xt0, ((0, 0), (0, 0), (0, 5)))
    wn1 = jnp.pad(wn1, ((0, 0), (0, 5)))
    wq1 = jnp.pad(wq1, ((0, 0), (0, 5)))

    idx1, p1, q1 = _edge_pre(xt0, wn1, wq1, b1)
    gm1_ = _sc_gather_max(idx1, p1.reshape(NB * N, -1), 64)

    idx2, p2, q2, avg1, mx1 = _edge_mid(gm1_.reshape(NB, N, -1), q1, wn2, wq2, b2)
    gm2_ = _sc_gather_max(idx2, p2.reshape(NB * N, -1), 64)

    idx3, p3, q3, avg2, mx2 = _edge_mid(gm2_.reshape(NB, N, -1), q2, wn3, wq3, b3)
    gm3_ = _sc_gather_max(idx3, p3.reshape(NB * N, -1), 128)

    idx4, p4, q4, avg3, mx3 = _edge_mid(gm3_.reshape(NB, N, -1), q3, wn4, wq4, b4)
    gm4_ = _sc_gather_max(idx4, p4.reshape(NB * N, -1), 256)

    avg4, mx4 = _tail(gm4_.reshape(NB, N, -1), q4)

    f = jnp.concatenate(
        [avg1[:, 0], avg2[:, 0], avg3[:, 0], avg4[:, 0],
         mx1[:, 0], mx2[:, 0], mx3[:, 0], mx4[:, 0]], axis=1)

    w1e = Wm1 * (gm1 * SCALE)[:, None]
    w2e = Wm2 * (gm2 * SCALE)[:, None]
    return _head(f, w1e, bm1, w2e, bm2, Wfc, bfc)
